# baseline (speedup 1.0000x reference)
import jax
import jax.numpy as jnp
from jax import lax
from jax.experimental import pallas as pl
from jax.experimental.pallas import tpu as pltpu

B, N, D = 4, 2048, 128
TJ = 1024
NJ = N // TJ


def _leaky(x):
    return jnp.maximum(x, 0.2 * x)


def _gat_kernel(x_ref, w_ref, asrc_ref, adst_ref, bias_ref, out_ref,
                xn_s, h_s, as_s, nad_s, u1_s, u2_s, v1_s, v2_s,
                acc_s, l_s):
    jt = pl.program_id(1)

    @pl.when(jt == 0)
    def _precompute():
        x = x_ref[0]
        x2 = x * x
        ones_d = jnp.ones((D, 1), dtype=jnp.float32)
        sq = jnp.dot(x2, ones_d, preferred_element_type=jnp.float32)
        inv = 1.0 / jnp.maximum(jnp.sqrt(sq), 1e-12)
        xn_s[...] = x * inv
        h = jnp.dot(x, w_ref[...], preferred_element_type=jnp.float32)
        h_s[...] = h
        a_s = jnp.dot(h, asrc_ref[...], preferred_element_type=jnp.float32)
        as_s[...] = a_s
        a_d = lax.dot_general(adst_ref[...], h, (((1,), (1,)), ((), ())),
                              preferred_element_type=jnp.float32)
        nad_s[...] = -a_d
        mh = _leaky(jnp.max(a_s) + a_d)
        u1_s[...] = jnp.exp(a_s)
        u2_s[...] = jnp.exp(0.2 * a_s)
        v1_s[...] = jnp.exp(a_d - mh)
        v2_s[...] = jnp.exp(0.2 * a_d - mh)

    xj = xn_s[pl.ds(jt * TJ, TJ), :]
    hj = h_s[pl.ds(jt * TJ, TJ), :]
    nad_j = nad_s[0, pl.ds(jt * TJ, TJ)][None, :]
    v1_j = v1_s[0, pl.ds(jt * TJ, TJ)][None, :]
    v2_j = v2_s[0, pl.ds(jt * TJ, TJ)][None, :]
    ones_col = jnp.ones((TJ, 1), dtype=jnp.float32)

    def _simtile(it):
        xi = xn_s[pl.ds(it * TJ, TJ), :]
        return lax.dot_general(xi, xj, (((1,), (1,)), ((), ())),
                               preferred_element_type=jnp.float32)

    def _weights(it):
        as_i = as_s[pl.ds(it * TJ, TJ), :]
        u1_i = u1_s[pl.ds(it * TJ, TJ), :]
        u2_i = u2_s[pl.ds(it * TJ, TJ), :]
        return jnp.where(as_i > nad_j, u1_i * v1_j, u2_i * v2_j)

    def _simbody(it, cm):
        sim = _simtile(it)
        return jnp.maximum(cm, jnp.max(sim, axis=0, keepdims=True))

    cm0 = jnp.full((1, TJ), -2.0, dtype=jnp.float32)
    cm = lax.fori_loop(0, jt, _simbody, cm0)
    cmax = jnp.max(cm)

    simd = _simtile(jt)
    cand = jnp.where(simd > 0.9, 1.0, 0.0)
    cnt = lax.dot_general(cand, ones_col, (((0,), (0,)), ((), ())),
                          preferred_element_type=jnp.float32)
    has_edge = jnp.logical_or(cmax > 0.9, jnp.max(cnt) > 1.5)

    @pl.when(jnp.logical_not(has_edge))
    def _selfloop_only():
        out_ref[0] = jnp.maximum(hj + bias_ref[...], 0.0)

    @pl.when(has_edge)
    def _aggregate():
        il = lax.broadcasted_iota(jnp.int32, (TJ, TJ), 0)
        jl = lax.broadcasted_iota(jnp.int32, (TJ, TJ), 1)
        keep = jnp.logical_and(simd > 0.9, il <= jl)
        exd = jnp.where(keep, _weights(jt), 0.0)
        acc_s[...] = lax.dot_general(exd, hj, (((0,), (0,)), ((), ())),
                                     preferred_element_type=jnp.float32)
        l_s[...] = lax.dot_general(exd, ones_col, (((0,), (0,)), ((), ())),
                                   preferred_element_type=jnp.float32)

        def _body(it, _):
            hi = h_s[pl.ds(it * TJ, TJ), :]
            sim = _simtile(it)
            ex = jnp.where(sim > 0.9, _weights(it), 0.0)
            acc_s[...] += lax.dot_general(ex, hi, (((0,), (0,)), ((), ())),
                                          preferred_element_type=jnp.float32)
            l_s[...] += lax.dot_general(ex, ones_col,
                                        (((0,), (0,)), ((), ())),
                                        preferred_element_type=jnp.float32)
            return 0

        lax.fori_loop(0, jt, _body, 0)

        out = acc_s[...] * (1.0 / l_s[...]) + bias_ref[...]
        out_ref[0] = jnp.maximum(out, 0.0)


@jax.jit
def kernel(distilled_features, W, att_src, att_dst, bias):
    asrc = att_src.reshape(D, 1)
    adst = att_dst.reshape(1, D)
    bias2 = bias.reshape(1, D)
    out = pl.pallas_call(
        _gat_kernel,
        grid=(B, NJ),
        in_specs=[
            pl.BlockSpec((1, N, D), lambda b, j: (b, 0, 0)),
            pl.BlockSpec((D, D), lambda b, j: (0, 0)),
            pl.BlockSpec((D, 1), lambda b, j: (0, 0)),
            pl.BlockSpec((1, D), lambda b, j: (0, 0)),
            pl.BlockSpec((1, D), lambda b, j: (0, 0)),
        ],
        out_specs=pl.BlockSpec((1, TJ, D), lambda b, j: (b, j, 0)),
        out_shape=jax.ShapeDtypeStruct((B, N, D), jnp.float32),
        scratch_shapes=[
            pltpu.VMEM((N, D), jnp.float32),
            pltpu.VMEM((N, D), jnp.float32),
            pltpu.VMEM((N, 1), jnp.float32),
            pltpu.VMEM((1, N), jnp.float32),
            pltpu.VMEM((N, 1), jnp.float32),
            pltpu.VMEM((N, 1), jnp.float32),
            pltpu.VMEM((1, N), jnp.float32),
            pltpu.VMEM((1, N), jnp.float32),
            pltpu.VMEM((TJ, D), jnp.float32),
            pltpu.VMEM((TJ, 1), jnp.float32),
        ],
        compiler_params=pltpu.CompilerParams(
            dimension_semantics=("arbitrary", "arbitrary"),
        ),
    )(distilled_features, W, asrc, adst, bias2)
    return out


# speedup vs baseline: 1.5493x; 1.5493x over previous
import jax
import jax.numpy as jnp
from jax import lax
from jax.experimental import pallas as pl
from jax.experimental.pallas import tpu as pltpu

B, N, D = 4, 2048, 128
TJ = 1024
NJ = N // TJ


def _leaky(x):
    return jnp.maximum(x, 0.2 * x)


def _gat_kernel(x_ref, w_ref, asrc_ref, adst_ref, bias_ref, out_ref,
                xn_s, h_s, as_s, nad_s, u1_s, u2_s, v1_s, v2_s,
                acc_s, l_s):
    jt = pl.program_id(1)

    @pl.when(jt == 0)
    def _precompute():
        x = x_ref[0]
        x2 = x * x
        ones_d = jnp.ones((D, 1), dtype=jnp.float32)
        sq = jnp.dot(x2, ones_d, preferred_element_type=jnp.float32)
        inv = 1.0 / jnp.maximum(jnp.sqrt(sq), 1e-12)
        xn_s[...] = x * inv
        h = jnp.dot(x, w_ref[...], preferred_element_type=jnp.float32)
        h_s[...] = h
        a_s = jnp.dot(h, asrc_ref[...], preferred_element_type=jnp.float32)
        as_s[...] = a_s
        a_d = lax.dot_general(adst_ref[...], h, (((1,), (1,)), ((), ())),
                              preferred_element_type=jnp.float32)
        nad_s[...] = -a_d
        mh = _leaky(jnp.max(a_s) + a_d)
        u1_s[...] = jnp.exp(a_s)
        u2_s[...] = jnp.exp(0.2 * a_s)
        v1_s[...] = jnp.exp(a_d - mh)
        v2_s[...] = jnp.exp(0.2 * a_d - mh)

    xj = xn_s[pl.ds(jt * TJ, TJ), :]
    hj = h_s[pl.ds(jt * TJ, TJ), :]
    nad_j = nad_s[0, pl.ds(jt * TJ, TJ)][None, :]
    v1_j = v1_s[0, pl.ds(jt * TJ, TJ)][None, :]
    v2_j = v2_s[0, pl.ds(jt * TJ, TJ)][None, :]
    ones_col = jnp.ones((TJ, 1), dtype=jnp.float32)

    def _simtile(it):
        xi = xn_s[pl.ds(it * TJ, TJ), :]
        return lax.dot_general(xi, xj, (((1,), (1,)), ((), ())),
                               preferred_element_type=jnp.float32)

    def _weights(it):
        as_i = as_s[pl.ds(it * TJ, TJ), :]
        u1_i = u1_s[pl.ds(it * TJ, TJ), :]
        u2_i = u2_s[pl.ds(it * TJ, TJ), :]
        return jnp.where(as_i > nad_j, u1_i * v1_j, u2_i * v2_j)

    def _simbody(it, cm):
        sim = _simtile(it)
        return jnp.maximum(cm, jnp.max(sim, axis=0, keepdims=True))

    cm0 = jnp.full((1, TJ), -2.0, dtype=jnp.float32)
    cm = lax.fori_loop(0, jt, _simbody, cm0)
    cmax = jnp.max(cm)

    simd = _simtile(jt)
    cand = jnp.where(simd > 0.9, 1.0, 0.0)
    cnt = jnp.sum(cand, axis=0, keepdims=True)
    has_edge = jnp.logical_or(cmax > 0.9, jnp.max(cnt) > 1.5)

    @pl.when(jnp.logical_not(has_edge))
    def _selfloop_only():
        out_ref[0] = jnp.maximum(hj + bias_ref[...], 0.0)

    @pl.when(has_edge)
    def _aggregate():
        il = lax.broadcasted_iota(jnp.int32, (TJ, TJ), 0)
        jl = lax.broadcasted_iota(jnp.int32, (TJ, TJ), 1)
        keep = jnp.logical_and(simd > 0.9, il <= jl)
        exd = jnp.where(keep, _weights(jt), 0.0)
        acc_s[...] = lax.dot_general(exd, hj, (((0,), (0,)), ((), ())),
                                     preferred_element_type=jnp.float32)
        l_s[...] = lax.dot_general(exd, ones_col, (((0,), (0,)), ((), ())),
                                   preferred_element_type=jnp.float32)

        def _body(it, _):
            hi = h_s[pl.ds(it * TJ, TJ), :]
            sim = _simtile(it)
            ex = jnp.where(sim > 0.9, _weights(it), 0.0)
            acc_s[...] += lax.dot_general(ex, hi, (((0,), (0,)), ((), ())),
                                          preferred_element_type=jnp.float32)
            l_s[...] += lax.dot_general(ex, ones_col,
                                        (((0,), (0,)), ((), ())),
                                        preferred_element_type=jnp.float32)
            return 0

        lax.fori_loop(0, jt, _body, 0)

        out = acc_s[...] * (1.0 / l_s[...]) + bias_ref[...]
        out_ref[0] = jnp.maximum(out, 0.0)


@jax.jit
def kernel(distilled_features, W, att_src, att_dst, bias):
    asrc = att_src.reshape(D, 1)
    adst = att_dst.reshape(1, D)
    bias2 = bias.reshape(1, D)
    out = pl.pallas_call(
        _gat_kernel,
        grid=(B, NJ),
        in_specs=[
            pl.BlockSpec((1, N, D), lambda b, j: (b, 0, 0)),
            pl.BlockSpec((D, D), lambda b, j: (0, 0)),
            pl.BlockSpec((D, 1), lambda b, j: (0, 0)),
            pl.BlockSpec((1, D), lambda b, j: (0, 0)),
            pl.BlockSpec((1, D), lambda b, j: (0, 0)),
        ],
        out_specs=pl.BlockSpec((1, TJ, D), lambda b, j: (b, j, 0)),
        out_shape=jax.ShapeDtypeStruct((B, N, D), jnp.float32),
        scratch_shapes=[
            pltpu.VMEM((N, D), jnp.float32),
            pltpu.VMEM((N, D), jnp.float32),
            pltpu.VMEM((N, 1), jnp.float32),
            pltpu.VMEM((1, N), jnp.float32),
            pltpu.VMEM((N, 1), jnp.float32),
            pltpu.VMEM((N, 1), jnp.float32),
            pltpu.VMEM((1, N), jnp.float32),
            pltpu.VMEM((1, N), jnp.float32),
            pltpu.VMEM((TJ, D), jnp.float32),
            pltpu.VMEM((TJ, 1), jnp.float32),
        ],
        compiler_params=pltpu.CompilerParams(
            dimension_semantics=("arbitrary", "arbitrary"),
        ),
    )(distilled_features, W, asrc, adst, bias2)
    return out


# speedup vs baseline: 1.6422x; 1.0600x over previous
import jax
import jax.numpy as jnp
from jax import lax
from jax.experimental import pallas as pl
from jax.experimental.pallas import tpu as pltpu

B, N, D = 4, 2048, 128
TJ = 1024
NJ = N // TJ


def _leaky(x):
    return jnp.maximum(x, 0.2 * x)


def _gat_kernel(x_ref, w_ref, asrc_ref, adst_ref, bias_ref, out_ref,
                xn_s, h_s, as_s, nad_s, u1_s, u2_s, v1_s, v2_s,
                acc_s, l_s):
    jt = pl.program_id(1)

    @pl.when(jt == 0)
    def _precompute():
        x = x_ref[0]
        x2 = x * x
        ones_d = jnp.ones((D, 1), dtype=jnp.float32)
        sq = jnp.dot(x2, ones_d, preferred_element_type=jnp.float32)
        inv = 1.0 / jnp.maximum(jnp.sqrt(sq), 1e-12)
        xn_s[...] = x * inv
        h = jnp.dot(x, w_ref[...], preferred_element_type=jnp.float32)
        h_s[...] = h
        a_s = jnp.dot(h, asrc_ref[...], preferred_element_type=jnp.float32)
        as_s[...] = a_s
        a_d = lax.dot_general(adst_ref[...], h, (((1,), (1,)), ((), ())),
                              preferred_element_type=jnp.float32)
        nad_s[...] = -a_d
        mh = _leaky(jnp.max(a_s) + a_d)
        u1_s[...] = jnp.exp(a_s)
        u2_s[...] = jnp.exp(0.2 * a_s)
        v1_s[...] = jnp.exp(a_d - mh)
        v2_s[...] = jnp.exp(0.2 * a_d - mh)

    xj = xn_s[pl.ds(jt * TJ, TJ), :]
    hj = h_s[pl.ds(jt * TJ, TJ), :]
    nad_j = nad_s[0, pl.ds(jt * TJ, TJ)][None, :]
    v1_j = v1_s[0, pl.ds(jt * TJ, TJ)][None, :]
    v2_j = v2_s[0, pl.ds(jt * TJ, TJ)][None, :]
    ones_col = jnp.ones((TJ, 1), dtype=jnp.float32)

    def _simtile(it):
        xi = xn_s[pl.ds(it * TJ, TJ), :]
        return lax.dot_general(xi, xj, (((1,), (1,)), ((), ())),
                               preferred_element_type=jnp.float32)

    def _weights(it):
        as_i = as_s[pl.ds(it * TJ, TJ), :]
        u1_i = u1_s[pl.ds(it * TJ, TJ), :]
        u2_i = u2_s[pl.ds(it * TJ, TJ), :]
        return jnp.where(as_i > nad_j, u1_i * v1_j, u2_i * v2_j)

    def _simbody(it, cm):
        sim = _simtile(it)
        return jnp.maximum(cm, jnp.max(sim, axis=0, keepdims=True))

    cm0 = jnp.full((1, TJ), -2.0, dtype=jnp.float32)
    cm = lax.fori_loop(0, jt, _simbody, cm0)
    cmax = jnp.max(cm)

    TH = TJ // 2
    xjA = xn_s[pl.ds(jt * TJ, TH), :]
    xjB = xn_s[pl.ds(jt * TJ + TH, TH), :]
    s1 = lax.dot_general(xjA, xjA, (((1,), (1,)), ((), ())),
                         preferred_element_type=jnp.float32)
    s2 = lax.dot_general(xjA, xjB, (((1,), (1,)), ((), ())),
                         preferred_element_type=jnp.float32)
    s3 = lax.dot_general(xjB, xjB, (((1,), (1,)), ((), ())),
                         preferred_element_type=jnp.float32)
    cnt1 = jnp.sum(jnp.where(s1 > 0.9, 1.0, 0.0), axis=0, keepdims=True)
    cnt3 = jnp.sum(jnp.where(s3 > 0.9, 1.0, 0.0), axis=0, keepdims=True)
    dmax = jnp.maximum(jnp.max(s2), cmax)
    dcnt = jnp.maximum(jnp.max(cnt1), jnp.max(cnt3))
    has_edge = jnp.logical_or(dmax > 0.9, dcnt > 1.5)

    @pl.when(jnp.logical_not(has_edge))
    def _selfloop_only():
        out_ref[0] = jnp.maximum(hj + bias_ref[...], 0.0)

    @pl.when(has_edge)
    def _aggregate():
        simd = _simtile(jt)
        il = lax.broadcasted_iota(jnp.int32, (TJ, TJ), 0)
        jl = lax.broadcasted_iota(jnp.int32, (TJ, TJ), 1)
        keep = jnp.logical_and(simd > 0.9, il <= jl)
        exd = jnp.where(keep, _weights(jt), 0.0)
        acc_s[...] = lax.dot_general(exd, hj, (((0,), (0,)), ((), ())),
                                     preferred_element_type=jnp.float32)
        l_s[...] = lax.dot_general(exd, ones_col, (((0,), (0,)), ((), ())),
                                   preferred_element_type=jnp.float32)

        def _body(it, _):
            hi = h_s[pl.ds(it * TJ, TJ), :]
            sim = _simtile(it)
            ex = jnp.where(sim > 0.9, _weights(it), 0.0)
            acc_s[...] += lax.dot_general(ex, hi, (((0,), (0,)), ((), ())),
                                          preferred_element_type=jnp.float32)
            l_s[...] += lax.dot_general(ex, ones_col,
                                        (((0,), (0,)), ((), ())),
                                        preferred_element_type=jnp.float32)
            return 0

        lax.fori_loop(0, jt, _body, 0)

        out = acc_s[...] * (1.0 / l_s[...]) + bias_ref[...]
        out_ref[0] = jnp.maximum(out, 0.0)


@jax.jit
def kernel(distilled_features, W, att_src, att_dst, bias):
    asrc = att_src.reshape(D, 1)
    adst = att_dst.reshape(1, D)
    bias2 = bias.reshape(1, D)
    out = pl.pallas_call(
        _gat_kernel,
        grid=(B, NJ),
        in_specs=[
            pl.BlockSpec((1, N, D), lambda b, j: (b, 0, 0)),
            pl.BlockSpec((D, D), lambda b, j: (0, 0)),
            pl.BlockSpec((D, 1), lambda b, j: (0, 0)),
            pl.BlockSpec((1, D), lambda b, j: (0, 0)),
            pl.BlockSpec((1, D), lambda b, j: (0, 0)),
        ],
        out_specs=pl.BlockSpec((1, TJ, D), lambda b, j: (b, j, 0)),
        out_shape=jax.ShapeDtypeStruct((B, N, D), jnp.float32),
        scratch_shapes=[
            pltpu.VMEM((N, D), jnp.float32),
            pltpu.VMEM((N, D), jnp.float32),
            pltpu.VMEM((N, 1), jnp.float32),
            pltpu.VMEM((1, N), jnp.float32),
            pltpu.VMEM((N, 1), jnp.float32),
            pltpu.VMEM((N, 1), jnp.float32),
            pltpu.VMEM((1, N), jnp.float32),
            pltpu.VMEM((1, N), jnp.float32),
            pltpu.VMEM((TJ, D), jnp.float32),
            pltpu.VMEM((TJ, 1), jnp.float32),
        ],
        compiler_params=pltpu.CompilerParams(
            dimension_semantics=("arbitrary", "arbitrary"),
        ),
    )(distilled_features, W, asrc, adst, bias2)
    return out
